# baseline (speedup 1.0000x reference)
import functools

import jax
import jax.numpy as jnp
from jax import lax
from jax.experimental import pallas as pl
from jax.experimental.pallas import tpu as pltpu
from jax.experimental.pallas import tpu_sc as plsc

N = 100000
D = 128
H = 128
NC = 16
ALPHA = 0.2

NUM_WORKERS = 32
CHUNK = 128
CHUNKS_PER_WORKER = 25
ROWS_PER_WORKER = CHUNK * CHUNKS_PER_WORKER
NP = NUM_WORKERS * ROWS_PER_WORKER

TC_ROWS = 2000


def _sc_gather(features, i1, i2):
  mesh = plsc.VectorSubcoreMesh(core_axis_name="c", subcore_axis_name="s")

  @functools.partial(
      pl.kernel,
      out_type=(
          jax.ShapeDtypeStruct((NP, D), jnp.float32),
          jax.ShapeDtypeStruct((NP, D), jnp.float32),
      ),
      mesh=mesh,
      scratch_types=[
          pltpu.VMEM((CHUNK,), jnp.int32),
          pltpu.VMEM((CHUNK, D), jnp.float32),
          pltpu.VMEM((CHUNK,), jnp.int32),
          pltpu.VMEM((CHUNK, D), jnp.float32),
          pltpu.SemaphoreType.DMA,
          pltpu.SemaphoreType.DMA,
      ],
  )
  def gather_kernel(f_hbm, i1_hbm, i2_hbm, g1_hbm, g2_hbm,
                    idx1_v, rows1_v, idx2_v, rows2_v, sem1, sem2):
    num_cores = 2
    wid = lax.axis_index("s") * num_cores + lax.axis_index("c")
    base = wid * ROWS_PER_WORKER

    def body(j, carry):
      off = pl.multiple_of(base + j * CHUNK, CHUNK)
      pltpu.sync_copy(i1_hbm.at[pl.ds(off, CHUNK)], idx1_v)
      pltpu.sync_copy(i2_hbm.at[pl.ds(off, CHUNK)], idx2_v)
      cp1 = pltpu.async_copy(f_hbm.at[idx1_v], rows1_v, sem1)
      cp2 = pltpu.async_copy(f_hbm.at[idx2_v], rows2_v, sem2)
      cp1.wait()
      pltpu.sync_copy(rows1_v, g1_hbm.at[pl.ds(off, CHUNK)])
      cp2.wait()
      pltpu.sync_copy(rows2_v, g2_hbm.at[pl.ds(off, CHUNK)])
      return carry

    lax.fori_loop(0, CHUNKS_PER_WORKER, body, 0)

  return gather_kernel(features, i1, i2)


def _tc_body(g1_ref, g2_ref, c_ref, w_ref, v_ref, o_ref, a_ref, b_ref):
  @pl.when(pl.program_id(0) == 0)
  def _fold_weights():
    cw1 = jnp.dot(c_ref[...], w_ref[:H, :], preferred_element_type=jnp.float32)
    cw2 = jnp.dot(c_ref[...], w_ref[H:, :], preferred_element_type=jnp.float32)
    a_ref[...] = jnp.dot(cw1, v_ref[...], preferred_element_type=jnp.float32)
    b_ref[...] = jnp.dot(cw2, v_ref[...], preferred_element_type=jnp.float32)

  outs = (jnp.dot(g1_ref[...], a_ref[...], preferred_element_type=jnp.float32)
          + jnp.dot(g2_ref[...], b_ref[...], preferred_element_type=jnp.float32))
  r = jnp.where(outs >= 0, outs, ALPHA * outs)
  m = jnp.max(r, axis=1, keepdims=True)
  shifted = r - m
  o_ref[...] = shifted - jnp.log(
      jnp.sum(jnp.exp(shifted), axis=1, keepdims=True))


def _tc_fused(g1, g2, C, W, V):
  grid = (N // TC_ROWS,)
  return pl.pallas_call(
      _tc_body,
      grid=grid,
      in_specs=[
          pl.BlockSpec((TC_ROWS, D), lambda i: (i, 0)),
          pl.BlockSpec((TC_ROWS, D), lambda i: (i, 0)),
          pl.BlockSpec((D, H), lambda i: (0, 0)),
          pl.BlockSpec((2 * H, H), lambda i: (0, 0)),
          pl.BlockSpec((H, NC), lambda i: (0, 0)),
      ],
      out_specs=pl.BlockSpec((TC_ROWS, NC), lambda i: (i, 0)),
      out_shape=jax.ShapeDtypeStruct((N, NC), jnp.float32),
      scratch_shapes=[
          pltpu.VMEM((H, NC), jnp.float32),
          pltpu.VMEM((H, NC), jnp.float32),
      ],
  )(g1, g2, C, W, V)


def kernel(features, C, W, V, n1, n2):
  pad = jnp.zeros((NP - N,), dtype=jnp.int32)
  i1 = jnp.concatenate([n1.astype(jnp.int32), pad])
  i2 = jnp.concatenate([n2.astype(jnp.int32), pad])
  g1, g2 = _sc_gather(features, i1, i2)
  return _tc_fused(g1, g2, C, W, V)


# speedup vs baseline: 1.1040x; 1.1040x over previous
import functools

import jax
import jax.numpy as jnp
from jax import lax
from jax.experimental import pallas as pl
from jax.experimental.pallas import tpu as pltpu
from jax.experimental.pallas import tpu_sc as plsc

N = 100000
D = 128
H = 128
NC = 16
ALPHA = 0.2

NUM_WORKERS = 32
CHUNK = 128
CHUNKS_PER_WORKER = 25
ROWS_PER_WORKER = CHUNK * CHUNKS_PER_WORKER
NP = NUM_WORKERS * ROWS_PER_WORKER
RING = 3

TC_ROWS = 2000


def _sc_gather(features, i1, i2):
  mesh = plsc.VectorSubcoreMesh(core_axis_name="c", subcore_axis_name="s")

  @functools.partial(
      pl.kernel,
      out_type=(
          jax.ShapeDtypeStruct((NP, D), jnp.float32),
          jax.ShapeDtypeStruct((NP, D), jnp.float32),
      ),
      mesh=mesh,
      scratch_types=(
          [
              pltpu.VMEM((CHUNKS_PER_WORKER, CHUNK), jnp.int32),
              pltpu.VMEM((CHUNKS_PER_WORKER, CHUNK), jnp.int32),
              pltpu.VMEM((RING, CHUNK, D), jnp.float32),
              pltpu.VMEM((RING, CHUNK, D), jnp.float32),
          ]
          + [pltpu.SemaphoreType.DMA] * (4 * RING)
      ),
  )
  def gather_kernel(f_hbm, i1_hbm, i2_hbm, g1_hbm, g2_hbm,
                    idx1_v, idx2_v, buf1, buf2, *sems):
    gs1 = sems[0:RING]
    gs2 = sems[RING:2 * RING]
    ws1 = sems[2 * RING:3 * RING]
    ws2 = sems[3 * RING:4 * RING]

    num_cores = 2
    wid = lax.axis_index("s") * num_cores + lax.axis_index("c")
    base = wid * ROWS_PER_WORKER

    pltpu.sync_copy(i1_hbm.at[wid], idx1_v)
    pltpu.sync_copy(i2_hbm.at[wid], idx2_v)

    g1cp = [None] * RING
    g2cp = [None] * RING
    w1cp = [None] * RING
    w2cp = [None] * RING

    for b in range(RING):
      g1cp[b] = pltpu.async_copy(f_hbm.at[idx1_v.at[b]], buf1.at[b], gs1[b])
      g2cp[b] = pltpu.async_copy(f_hbm.at[idx2_v.at[b]], buf2.at[b], gs2[b])

    for j in range(CHUNKS_PER_WORKER):
      b = j % RING
      off = base + j * CHUNK
      g1cp[b].wait()
      w1cp[b] = pltpu.async_copy(buf1.at[b], g1_hbm.at[pl.ds(off, CHUNK)],
                                 ws1[b])
      g2cp[b].wait()
      w2cp[b] = pltpu.async_copy(buf2.at[b], g2_hbm.at[pl.ds(off, CHUNK)],
                                 ws2[b])
      nj = j + RING
      if nj < CHUNKS_PER_WORKER:
        w1cp[b].wait()
        g1cp[b] = pltpu.async_copy(f_hbm.at[idx1_v.at[nj]], buf1.at[b],
                                   gs1[b])
        w2cp[b].wait()
        g2cp[b] = pltpu.async_copy(f_hbm.at[idx2_v.at[nj]], buf2.at[b],
                                   gs2[b])
      else:
        w1cp[b].wait()
        w2cp[b].wait()

  return gather_kernel(features, i1, i2)


def _tc_body(g1_ref, g2_ref, c_ref, w_ref, v_ref, o_ref, a_ref, b_ref):
  @pl.when(pl.program_id(0) == 0)
  def _fold_weights():
    cw1 = jnp.dot(c_ref[...], w_ref[:H, :], preferred_element_type=jnp.float32)
    cw2 = jnp.dot(c_ref[...], w_ref[H:, :], preferred_element_type=jnp.float32)
    a_ref[...] = jnp.dot(cw1, v_ref[...], preferred_element_type=jnp.float32)
    b_ref[...] = jnp.dot(cw2, v_ref[...], preferred_element_type=jnp.float32)

  outs = (jnp.dot(g1_ref[...], a_ref[...], preferred_element_type=jnp.float32)
          + jnp.dot(g2_ref[...], b_ref[...], preferred_element_type=jnp.float32))
  r = jnp.where(outs >= 0, outs, ALPHA * outs)
  m = jnp.max(r, axis=1, keepdims=True)
  shifted = r - m
  o_ref[...] = shifted - jnp.log(
      jnp.sum(jnp.exp(shifted), axis=1, keepdims=True))


def _tc_fused(g1, g2, C, W, V):
  grid = (N // TC_ROWS,)
  return pl.pallas_call(
      _tc_body,
      grid=grid,
      in_specs=[
          pl.BlockSpec((TC_ROWS, D), lambda i: (i, 0)),
          pl.BlockSpec((TC_ROWS, D), lambda i: (i, 0)),
          pl.BlockSpec((D, H), lambda i: (0, 0)),
          pl.BlockSpec((2 * H, H), lambda i: (0, 0)),
          pl.BlockSpec((H, NC), lambda i: (0, 0)),
      ],
      out_specs=pl.BlockSpec((TC_ROWS, NC), lambda i: (i, 0)),
      out_shape=jax.ShapeDtypeStruct((N, NC), jnp.float32),
      scratch_shapes=[
          pltpu.VMEM((H, NC), jnp.float32),
          pltpu.VMEM((H, NC), jnp.float32),
      ],
  )(g1, g2, C, W, V)


def kernel(features, C, W, V, n1, n2):
  pad = jnp.zeros((NP - N,), dtype=jnp.int32)
  i1 = jnp.concatenate([n1.astype(jnp.int32), pad]).reshape(
      NUM_WORKERS, CHUNKS_PER_WORKER, CHUNK)
  i2 = jnp.concatenate([n2.astype(jnp.int32), pad]).reshape(
      NUM_WORKERS, CHUNKS_PER_WORKER, CHUNK)
  g1, g2 = _sc_gather(features, i1, i2)
  return _tc_fused(g1, g2, C, W, V)


# speedup vs baseline: 1.1411x; 1.0336x over previous
import functools

import jax
import jax.numpy as jnp
from jax import lax
from jax.experimental import pallas as pl
from jax.experimental.pallas import tpu as pltpu
from jax.experimental.pallas import tpu_sc as plsc

N = 100000
D = 128
H = 128
NC = 16
ALPHA = 0.2

NUM_WORKERS = 32
CHUNK = 128
CHUNKS_PER_WORKER = 25
ROWS_PER_WORKER = CHUNK * CHUNKS_PER_WORKER
NP = NUM_WORKERS * ROWS_PER_WORKER
RING = 3

TC_ROWS = 4000


def _sc_gather(features, i1, i2):
  mesh = plsc.VectorSubcoreMesh(core_axis_name="c", subcore_axis_name="s")

  @functools.partial(
      pl.kernel,
      out_type=(
          jax.ShapeDtypeStruct((NP, D), jnp.float32),
          jax.ShapeDtypeStruct((NP, D), jnp.float32),
      ),
      mesh=mesh,
      scratch_types=(
          [
              pltpu.VMEM((CHUNKS_PER_WORKER, CHUNK), jnp.int32),
              pltpu.VMEM((CHUNKS_PER_WORKER, CHUNK), jnp.int32),
              pltpu.VMEM((RING, CHUNK, D), jnp.float32),
              pltpu.VMEM((RING, CHUNK, D), jnp.float32),
          ]
          + [pltpu.SemaphoreType.DMA] * (4 * RING)
      ),
  )
  def gather_kernel(f_hbm, i1_hbm, i2_hbm, g1_hbm, g2_hbm,
                    idx1_v, idx2_v, buf1, buf2, *sems):
    gs1 = sems[0:RING]
    gs2 = sems[RING:2 * RING]
    ws1 = sems[2 * RING:3 * RING]
    ws2 = sems[3 * RING:4 * RING]

    num_subcores = 16
    wid = lax.axis_index("c") * num_subcores + lax.axis_index("s")
    base = wid * ROWS_PER_WORKER

    pltpu.sync_copy(i1_hbm.at[wid], idx1_v)
    pltpu.sync_copy(i2_hbm.at[wid], idx2_v)

    g1cp = [None] * RING
    g2cp = [None] * RING
    w1cp = [None] * RING
    w2cp = [None] * RING

    for b in range(RING):
      g1cp[b] = pltpu.async_copy(f_hbm.at[idx1_v.at[b]], buf1.at[b], gs1[b])
      g2cp[b] = pltpu.async_copy(f_hbm.at[idx2_v.at[b]], buf2.at[b], gs2[b])

    for j in range(CHUNKS_PER_WORKER):
      b = j % RING
      off = base + j * CHUNK
      g1cp[b].wait()
      w1cp[b] = pltpu.async_copy(buf1.at[b], g1_hbm.at[pl.ds(off, CHUNK)],
                                 ws1[b])
      g2cp[b].wait()
      w2cp[b] = pltpu.async_copy(buf2.at[b], g2_hbm.at[pl.ds(off, CHUNK)],
                                 ws2[b])
      nj = j + RING
      if nj < CHUNKS_PER_WORKER:
        w1cp[b].wait()
        g1cp[b] = pltpu.async_copy(f_hbm.at[idx1_v.at[nj]], buf1.at[b],
                                   gs1[b])
        w2cp[b].wait()
        g2cp[b] = pltpu.async_copy(f_hbm.at[idx2_v.at[nj]], buf2.at[b],
                                   gs2[b])
      else:
        w1cp[b].wait()
        w2cp[b].wait()

  return gather_kernel(features, i1, i2)


def _tc_body(g1_ref, g2_ref, c_ref, w_ref, v_ref, o_ref, a_ref, b_ref):
  @pl.when(pl.program_id(0) == 0)
  def _fold_weights():
    cw1 = jnp.dot(c_ref[...], w_ref[:H, :], preferred_element_type=jnp.float32)
    cw2 = jnp.dot(c_ref[...], w_ref[H:, :], preferred_element_type=jnp.float32)
    a_ref[...] = jnp.dot(cw1, v_ref[...], preferred_element_type=jnp.float32)
    b_ref[...] = jnp.dot(cw2, v_ref[...], preferred_element_type=jnp.float32)

  outs = (jnp.dot(g1_ref[...], a_ref[...], preferred_element_type=jnp.float32)
          + jnp.dot(g2_ref[...], b_ref[...], preferred_element_type=jnp.float32))
  r = jnp.where(outs >= 0, outs, ALPHA * outs)
  m = jnp.max(r, axis=1, keepdims=True)
  shifted = r - m
  o_ref[...] = shifted - jnp.log(
      jnp.sum(jnp.exp(shifted), axis=1, keepdims=True))


def _tc_fused(g1, g2, C, W, V):
  grid = (N // TC_ROWS,)
  return pl.pallas_call(
      _tc_body,
      grid=grid,
      in_specs=[
          pl.BlockSpec((TC_ROWS, D), lambda i: (i, 0)),
          pl.BlockSpec((TC_ROWS, D), lambda i: (i, 0)),
          pl.BlockSpec((D, H), lambda i: (0, 0)),
          pl.BlockSpec((2 * H, H), lambda i: (0, 0)),
          pl.BlockSpec((H, NC), lambda i: (0, 0)),
      ],
      out_specs=pl.BlockSpec((TC_ROWS, NC), lambda i: (i, 0)),
      out_shape=jax.ShapeDtypeStruct((N, NC), jnp.float32),
      scratch_shapes=[
          pltpu.VMEM((H, NC), jnp.float32),
          pltpu.VMEM((H, NC), jnp.float32),
      ],
  )(g1, g2, C, W, V)


def kernel(features, C, W, V, n1, n2):
  pad = jnp.zeros((NP - N,), dtype=jnp.int32)
  i1 = jnp.concatenate([n1.astype(jnp.int32), pad]).reshape(
      NUM_WORKERS, CHUNKS_PER_WORKER, CHUNK)
  i2 = jnp.concatenate([n2.astype(jnp.int32), pad]).reshape(
      NUM_WORKERS, CHUNKS_PER_WORKER, CHUNK)
  g1, g2 = _sc_gather(features, i1, i2)
  return _tc_fused(g1, g2, C, W, V)


# speedup vs baseline: 1.1779x; 1.0322x over previous
import functools

import jax
import jax.numpy as jnp
from jax import lax
from jax.experimental import pallas as pl
from jax.experimental.pallas import tpu as pltpu
from jax.experimental.pallas import tpu_sc as plsc

N = 100000
D = 128
H = 128
NC = 16
ALPHA = 0.2

NUM_WORKERS = 32
CHUNK = 128
TOTAL_CHUNKS = 800
NP = TOTAL_CHUNKS * CHUNK
K0 = 42
K1 = 8
RING = 3

TC_ROWS = 4000


def _sc_gather(features, i1, i2):
  mesh = plsc.VectorSubcoreMesh(core_axis_name="c", subcore_axis_name="s")

  @functools.partial(
      pl.kernel,
      out_type=(
          jax.ShapeDtypeStruct((NP, D), jnp.float32),
          jax.ShapeDtypeStruct((NP, D), jnp.float32),
      ),
      mesh=mesh,
      scratch_types=[
          pltpu.VMEM((K0, CHUNK), jnp.int32),
          pltpu.VMEM((K0, CHUNK), jnp.int32),
          pltpu.VMEM((RING, CHUNK, D), jnp.float32),
          pltpu.VMEM((RING, CHUNK, D), jnp.float32),
          pltpu.SemaphoreType.DMA((RING,)),
          pltpu.SemaphoreType.DMA((RING,)),
          pltpu.SemaphoreType.DMA((RING,)),
          pltpu.SemaphoreType.DMA((RING,)),
      ],
  )
  def gather_kernel(f_hbm, i1_hbm, i2_hbm, g1_hbm, g2_hbm,
                    idx1_v, idx2_v, buf1, buf2, gs1, gs2, ws1, ws2):
    cid = lax.axis_index("c")
    sid = lax.axis_index("s")
    num_subcores = 16
    wid = cid * num_subcores + sid
    kcount = jnp.where(cid == 0, K0, K1)
    cstart = jnp.where(cid == 0, sid * K0, num_subcores * K0 + sid * K1)

    pltpu.sync_copy(i1_hbm.at[wid], idx1_v)
    pltpu.sync_copy(i2_hbm.at[wid], idx2_v)

    def fire_gather(k, b):
      pltpu.async_copy(f_hbm.at[idx1_v.at[k]], buf1.at[b], gs1.at[b])
      pltpu.async_copy(f_hbm.at[idx2_v.at[k]], buf2.at[b], gs2.at[b])

    for b in range(RING):
      fire_gather(b, b)

    def wait_gather(b):
      pltpu.make_async_copy(f_hbm.at[pl.ds(0, CHUNK)], buf1.at[b],
                            gs1.at[b]).wait()
      pltpu.make_async_copy(f_hbm.at[pl.ds(0, CHUNK)], buf2.at[b],
                            gs2.at[b]).wait()

    def wait_write(b):
      pltpu.make_async_copy(buf1.at[b], g1_hbm.at[pl.ds(0, CHUNK)],
                            ws1.at[b]).wait()
      pltpu.make_async_copy(buf2.at[b], g2_hbm.at[pl.ds(0, CHUNK)],
                            ws2.at[b]).wait()

    def body(j, carry):
      b = lax.rem(j, RING)
      off = pl.multiple_of((cstart + j) * CHUNK, CHUNK)
      pltpu.make_async_copy(f_hbm.at[pl.ds(0, CHUNK)], buf1.at[b],
                            gs1.at[b]).wait()
      pltpu.async_copy(buf1.at[b], g1_hbm.at[pl.ds(off, CHUNK)], ws1.at[b])
      pltpu.make_async_copy(f_hbm.at[pl.ds(0, CHUNK)], buf2.at[b],
                            gs2.at[b]).wait()
      pltpu.async_copy(buf2.at[b], g2_hbm.at[pl.ds(off, CHUNK)], ws2.at[b])

      @pl.when(j + RING < kcount)
      def _refill():
        wait_write(b)
        fire_gather(j + RING, b)

      return carry

    lax.fori_loop(0, kcount, body, 0)

    for b in range(RING):
      wait_write(b)

  return gather_kernel(features, i1, i2)


def _tc_body(g1_ref, g2_ref, c_ref, w_ref, v_ref, o_ref, a_ref, b_ref):
  @pl.when(pl.program_id(0) == 0)
  def _fold_weights():
    cw1 = jnp.dot(c_ref[...], w_ref[:H, :], preferred_element_type=jnp.float32)
    cw2 = jnp.dot(c_ref[...], w_ref[H:, :], preferred_element_type=jnp.float32)
    a_ref[...] = jnp.dot(cw1, v_ref[...], preferred_element_type=jnp.float32)
    b_ref[...] = jnp.dot(cw2, v_ref[...], preferred_element_type=jnp.float32)

  outs = (jnp.dot(g1_ref[...], a_ref[...], preferred_element_type=jnp.float32)
          + jnp.dot(g2_ref[...], b_ref[...], preferred_element_type=jnp.float32))
  r = jnp.where(outs >= 0, outs, ALPHA * outs)
  m = jnp.max(r, axis=1, keepdims=True)
  shifted = r - m
  o_ref[...] = shifted - jnp.log(
      jnp.sum(jnp.exp(shifted), axis=1, keepdims=True))


def _tc_fused(g1, g2, C, W, V):
  grid = (N // TC_ROWS,)
  return pl.pallas_call(
      _tc_body,
      grid=grid,
      in_specs=[
          pl.BlockSpec((TC_ROWS, D), lambda i: (i, 0)),
          pl.BlockSpec((TC_ROWS, D), lambda i: (i, 0)),
          pl.BlockSpec((D, H), lambda i: (0, 0)),
          pl.BlockSpec((2 * H, H), lambda i: (0, 0)),
          pl.BlockSpec((H, NC), lambda i: (0, 0)),
      ],
      out_specs=pl.BlockSpec((TC_ROWS, NC), lambda i: (i, 0)),
      out_shape=jax.ShapeDtypeStruct((N, NC), jnp.float32),
      scratch_shapes=[
          pltpu.VMEM((H, NC), jnp.float32),
          pltpu.VMEM((H, NC), jnp.float32),
      ],
  )(g1, g2, C, W, V)


def kernel(features, C, W, V, n1, n2):
  def pack(idx):
    pad = jnp.zeros((NP - N,), dtype=jnp.int32)
    chunks = jnp.concatenate([idx.astype(jnp.int32), pad]).reshape(
        TOTAL_CHUNKS, CHUNK)
    c0 = chunks[:16 * K0].reshape(16, K0, CHUNK)
    c1 = chunks[16 * K0:].reshape(16, K1, CHUNK)
    c1 = jnp.pad(c1, ((0, 0), (0, K0 - K1), (0, 0)))
    return jnp.concatenate([c0, c1], axis=0)

  g1, g2 = _sc_gather(features, pack(n1), pack(n2))
  return _tc_fused(g1, g2, C, W, V)
